# baseline (speedup 1.0000x reference)
import jax
import jax.numpy as jnp
from jax import lax
from jax.experimental import pallas as pl
from jax.experimental.pallas import tpu as pltpu
from jax.experimental.pallas import tpu_sc as plsc

_N = 4096
_D = 512
_K = 10
_KP = 16
_NC = 64
_NKM = 4
_NITER = 20
_RB = 256
_SENT = 2 ** 30

_NW = 32
_RPW = _N // _NW
_GCH = _RPW * _KP // 128
_ZPW = _N * _K // _NW
_ZCH = _ZPW // 128
_RSTR = 1344
_ZOFF = 1296
_OPW = _N * _K // _NW


def _simtopk_body(s_ref, t_ref, o_ref):
    i = pl.program_id(0)
    sim = lax.dot_general(s_ref[...], t_ref[...], (((1,), (1,)), ((), ())),
                          preferred_element_type=jnp.float32)
    row_g = i * _RB + lax.broadcasted_iota(jnp.int32, (_RB, _N), 0)
    col = lax.broadcasted_iota(jnp.int32, (_RB, _N), 1)
    sim = sim + jnp.where(col == row_g, jnp.float32(10.0), jnp.float32(0.0))
    idxs = []
    for _ in range(_K):
        idx = jnp.argmax(sim, axis=1).astype(jnp.int32)
        idxs.append(idx)
        sim = jnp.where(col == idx[:, None], -jnp.inf, sim)
    pad = jnp.full((_RB, _KP - _K), _SENT, jnp.int32)
    o_ref[...] = jnp.concatenate([jnp.stack(idxs, axis=1), pad], axis=1)


def _kmeans_body(x_ref, xt_ref, c_ref, o_ref):
    x = x_ref[...]
    xt = xt_ref[...]
    xx = jnp.sum(x * x, axis=1, keepdims=True)

    def _d2(centt):
        ip = lax.dot_general(x, centt, (((1,), (0,)), ((), ())),
                             preferred_element_type=jnp.float32)
        cc = jnp.sum(centt * centt, axis=0)
        return (xx - 2.0 * ip) + cc[None, :]

    def _step(_, centt):
        d2 = _d2(centt)
        h = jnp.concatenate(
            [(d2[:, s * _NC:(s + 1) * _NC]
              == jnp.min(d2[:, s * _NC:(s + 1) * _NC], axis=1, keepdims=True)
              ).astype(jnp.float32) for s in range(_NKM)], axis=1)
        sums_t = lax.dot_general(xt, h, (((1,), (0,)), ((), ())),
                                 preferred_element_type=jnp.float32)
        cnt = sums_t[_D, :]
        return sums_t[:_D, :] / jnp.clip(cnt, 1.0)[None, :]

    centt = lax.fori_loop(0, _NITER, _step, c_ref[...])
    d2 = _d2(centt)
    for s in range(_NKM):
        o_ref[s, :] = jnp.argmin(
            d2[:, s * _NC:(s + 1) * _NC], axis=1).astype(jnp.int32)


def _s1_body(adj_ref, knn_ref, lab_ref, winfo_ref, interc_out, interr_out,
             tot_out, knn_v, lab_v, gidx_v, adjv_v, cstg_v, rstg_v, widb_v,
             tot_v, sem_g):
    wid = lax.axis_index("s") * 2 + lax.axis_index("c")
    base_row = wid * _RPW
    pltpu.sync_copy(knn_ref.at[pl.ds(base_row, _RPW)], knn_v)
    pltpu.sync_copy(winfo_ref.at[wid], widb_v)
    widv = widb_v[...]
    base_rowv = widv * _RPW

    def _gbody(r, gb):
        cols = knn_v[r]
        valid = cols < _N
        g = jnp.where(valid, gb + cols, 0)
        gidx_v[r // 8, pl.ds((r % 8) * _KP, _KP)] = g
        return gb + _N
    lax.fori_loop(0, _RPW, _gbody, base_rowv * _N)

    pend = [pltpu.async_copy(adj_ref.at[gidx_v.at[j]], adjv_v.at[j], sem_g)
            for j in range(_GCH)]
    pltpu.sync_copy(lab_ref.at[pl.ds(0, _NKM * _N)], lab_v)

    def _zfill(q, c):
        cstg_v[pl.ds(_ZOFF + q * _KP, _KP)] = jnp.zeros((_KP,), jnp.int32)
        rstg_v[pl.ds(_ZOFF + q * _KP, _KP)] = jnp.zeros((_KP,), jnp.int32)
        return c
    lax.fori_loop(0, (_RSTR - _ZOFF) // _KP, _zfill, jnp.int32(0))
    for dsc in pend:
        dsc.wait()

    def _rbody(r, carry):
        rgv, totv = carry
        cols = knn_v[r]
        valid = cols < _N
        av = adjv_v[r // 8, pl.ds((r % 8) * _KP, _KP)]
        m = av != 0.0
        safe = jnp.where(valid, cols, 0)
        for s in range(_NKM):
            own = plsc.load_gather(lab_v, [rgv + (s * _N)])
            gl = plsc.load_gather(lab_v, [safe + (s * _N)])
            m = m | (gl == own)
        keep = valid & m
        key = jnp.where(keep, cols, jnp.int32(_SENT))
        skey, _ = plsc.sort_key_val(key, cols)
        cntv = plsc.all_reduce_population_count(keep)
        off = totv[0]
        cstg_v[pl.ds(off, _KP)] = skey
        rstg_v[pl.ds(off, _KP)] = rgv
        return (rgv + 1, totv + cntv)
    _, totv = lax.fori_loop(
        0, _RPW, _rbody, (base_rowv, jnp.zeros((_KP,), jnp.int32)))

    tot_v[...] = totv
    pltpu.sync_copy(cstg_v, interc_out.at[pl.ds(wid * _RSTR, _RSTR)])
    pltpu.sync_copy(rstg_v, interr_out.at[pl.ds(wid * _RSTR, _RSTR)])
    pltpu.sync_copy(tot_v, tot_out.at[wid])


def _srcidx_body(t_ref, o_ref):
    ti = t_ref[...]
    b = jnp.int32(0)
    bases = []
    for w in range(_NW):
        bases.append(b)
        b = b + ti[w, 0]
    m = b
    rows = lax.broadcasted_iota(jnp.int32, (_NW * _KP, 128), 0)
    cols = lax.broadcasted_iota(jnp.int32, (_NW * _KP, 128), 1)
    w16 = rows // _KP
    p = w16 * _OPW + (rows % _KP) * 128 + cols
    acc = p
    for v in range(_NW - 1):
        acc = acc + (p >= bases[v + 1]).astype(jnp.int32) * (_RSTR - ti[v, 0])
    zsrc = w16 * _RSTR + _ZOFF + (cols % 32)
    o_ref[...] = jnp.where(p >= m, zsrc, acc)


def _s2_body(interc_ref, interr_ref, src_ref, rows_out, cols_out,
             gidx_v, cval_v, rval_v, sem_s):
    wid = lax.axis_index("s") * 2 + lax.axis_index("c")
    pltpu.sync_copy(src_ref.at[pl.ds(wid * _KP, _KP)], gidx_v)

    pend = []
    for j in range(_OPW // 128):
        pend.append(pltpu.async_copy(
            interc_ref.at[gidx_v.at[j]], cval_v.at[j], sem_s))
        pend.append(pltpu.async_copy(
            interr_ref.at[gidx_v.at[j]], rval_v.at[j], sem_s))
        if len(pend) == 8:
            for dsc in pend:
                dsc.wait()
            pend = []
    for dsc in pend:
        dsc.wait()

    pltpu.sync_copy(cval_v, cols_out.at[wid])
    pltpu.sync_copy(rval_v, rows_out.at[wid])


def _run_simtopk(student, teacher):
    return pl.pallas_call(
        _simtopk_body,
        grid=(_N // _RB,),
        in_specs=[pl.BlockSpec((_RB, _D), lambda i: (i, 0)),
                  pl.BlockSpec((_N, _D), lambda i: (0, 0))],
        out_specs=pl.BlockSpec((_RB, _KP), lambda i: (i, 0)),
        out_shape=jax.ShapeDtypeStruct((_N, _KP), jnp.int32),
    )(student, teacher)


def _run_kmeans(teacher, teacher_t, cent0t):
    return pl.pallas_call(
        _kmeans_body,
        out_shape=jax.ShapeDtypeStruct((8, _N), jnp.int32),
    )(teacher, teacher_t, cent0t)


_SC_MESH = dict(core_axis_name="c", subcore_axis_name="s")


def _run_s1(adj_flat, knn, labs_flat, winfo):
    fn = pl.kernel(
        _s1_body,
        out_type=[jax.ShapeDtypeStruct((_NW * _RSTR,), jnp.int32),
                  jax.ShapeDtypeStruct((_NW * _RSTR,), jnp.int32),
                  jax.ShapeDtypeStruct((_NW, _KP), jnp.int32)],
        mesh=plsc.VectorSubcoreMesh(**_SC_MESH),
        compiler_params=pltpu.CompilerParams(needs_layout_passes=False),
        scratch_types=[
            pltpu.VMEM((_RPW, _KP), jnp.int32),
            pltpu.VMEM((_NKM * _N,), jnp.int32),
            pltpu.VMEM((_GCH, 128), jnp.int32),
            pltpu.VMEM((_GCH, 128), jnp.float32),
            pltpu.VMEM((_RSTR,), jnp.int32),
            pltpu.VMEM((_RSTR,), jnp.int32),
            pltpu.VMEM((_KP,), jnp.int32),
            pltpu.VMEM((_KP,), jnp.int32),
            pltpu.SemaphoreType.DMA,
        ],
    )
    return fn(adj_flat, knn, labs_flat, winfo)


def _run_srcidx(tots):
    return pl.pallas_call(
        _srcidx_body,
        out_shape=jax.ShapeDtypeStruct((_NW * _KP, 128), jnp.int32),
    )(tots)


def _run_s2(interc, interr, srcidx):
    fn = pl.kernel(
        _s2_body,
        out_type=[jax.ShapeDtypeStruct((_NW, _OPW // 128, 128), jnp.int32),
                  jax.ShapeDtypeStruct((_NW, _OPW // 128, 128), jnp.int32)],
        mesh=plsc.VectorSubcoreMesh(**_SC_MESH),
        compiler_params=pltpu.CompilerParams(needs_layout_passes=False),
        scratch_types=[
            pltpu.VMEM((_KP, 128), jnp.int32),
            pltpu.VMEM((_OPW // 128, 128), jnp.int32),
            pltpu.VMEM((_OPW // 128, 128), jnp.int32),
            pltpu.SemaphoreType.DMA,
        ],
    )
    return fn(interc, interr, srcidx)


def kernel(adj, student, teacher, top_k):
    tsg = lax.stop_gradient(teacher)
    knn = _run_simtopk(student, tsg)

    cents = []
    for s in range(_NKM):
        kk = jax.random.key(1234 + s)
        ii = jax.random.choice(kk, _N, shape=(_NC,), replace=False)
        cents.append(tsg[ii])
    cent0 = jnp.concatenate(cents, axis=0)
    xt_ext = jnp.concatenate(
        [tsg.T, jnp.ones((1, _N), jnp.float32)], axis=0)
    labs = _run_kmeans(tsg, xt_ext, cent0.T)

    winfo = jnp.broadcast_to(
        jnp.arange(_NW, dtype=jnp.int32)[:, None], (_NW, _KP))
    interc, interr, tots = _run_s1(adj.reshape(-1), knn, labs.reshape(-1),
                                   winfo)
    srcidx = _run_srcidx(tots)
    rows_o, cols_o = _run_s2(interc, interr, srcidx)
    indices = jnp.stack([rows_o.reshape(-1), cols_o.reshape(-1)], axis=0)
    return (indices, top_k)


# speedup vs baseline: 1.0190x; 1.0190x over previous
import jax
import jax.numpy as jnp
from jax import lax
from jax.experimental import pallas as pl
from jax.experimental.pallas import tpu as pltpu
from jax.experimental.pallas import tpu_sc as plsc

_N = 4096
_D = 512
_K = 10
_KP = 16
_NC = 64
_NKM = 4
_NITER = 20
_RB = 256
_SENT = 2 ** 30

_NW = 32
_RPW = _N // _NW
_GCH = _RPW * _KP // 128
_ZPW = _N * _K // _NW
_ZCH = _ZPW // 128
_RSTR = 1344
_ZOFF = 1296
_OPW = _N * _K // _NW


def _simtopk_body(s_ref, t_ref, o_ref):
    i = pl.program_id(0)
    sim = lax.dot_general(s_ref[...], t_ref[...], (((1,), (1,)), ((), ())),
                          preferred_element_type=jnp.float32)
    row_g = i * _RB + lax.broadcasted_iota(jnp.int32, (_RB, _N), 0)
    col = lax.broadcasted_iota(jnp.int32, (_RB, _N), 1)
    sim = sim + jnp.where(col == row_g, jnp.float32(10.0), jnp.float32(0.0))
    idxs = []
    for _ in range(_K):
        idx = jnp.argmax(sim, axis=1).astype(jnp.int32)
        idxs.append(idx)
        sim = jnp.where(col == idx[:, None], -jnp.inf, sim)
    pad = jnp.full((_RB, _KP - _K), _SENT, jnp.int32)
    o_ref[...] = jnp.concatenate([jnp.stack(idxs, axis=1), pad], axis=1)


def _kmeans_body(x_ref, xt_ref, c_ref, o_ref):
    x = x_ref[...]
    xt = xt_ref[...]
    xx = jnp.sum(x * x, axis=1, keepdims=True)

    def _d2(centt):
        ip = lax.dot_general(x, centt, (((1,), (0,)), ((), ())),
                             preferred_element_type=jnp.float32)
        cc = jnp.sum(centt * centt, axis=0)
        return (xx - 2.0 * ip) + cc[None, :]

    def _step(_, centt):
        d2 = _d2(centt)
        h = jnp.concatenate(
            [(d2[:, s * _NC:(s + 1) * _NC]
              == jnp.min(d2[:, s * _NC:(s + 1) * _NC], axis=1, keepdims=True)
              ).astype(jnp.float32) for s in range(_NKM)], axis=1)
        sums_t = lax.dot_general(xt, h, (((1,), (0,)), ((), ())),
                                 preferred_element_type=jnp.float32)
        cnt = jnp.sum(h, axis=0)
        return sums_t / jnp.clip(cnt, 1.0)[None, :]

    centt = lax.fori_loop(0, _NITER, _step, c_ref[...])
    d2 = _d2(centt)
    for s in range(_NKM):
        o_ref[s, :] = jnp.argmin(
            d2[:, s * _NC:(s + 1) * _NC], axis=1).astype(jnp.int32)


def _s1_body(adj_ref, knn_ref, lab_ref, winfo_ref, interc_out, interr_out,
             tot_out, knn_v, lab_v, gidx_v, adjv_v, cstg_v, rstg_v, widb_v,
             tot_v, sem_g):
    wid = lax.axis_index("s") * 2 + lax.axis_index("c")
    base_row = wid * _RPW
    pltpu.sync_copy(knn_ref.at[pl.ds(base_row, _RPW)], knn_v)
    pltpu.sync_copy(winfo_ref.at[wid], widb_v)
    widv = widb_v[...]
    base_rowv = widv * _RPW

    def _gbody(r, gb):
        cols = knn_v[r]
        valid = cols < _N
        g = jnp.where(valid, gb + cols, 0)
        gidx_v[r // 8, pl.ds((r % 8) * _KP, _KP)] = g
        return gb + _N
    lax.fori_loop(0, _RPW, _gbody, base_rowv * _N)

    pend = [pltpu.async_copy(adj_ref.at[gidx_v.at[j]], adjv_v.at[j], sem_g)
            for j in range(_GCH)]
    pltpu.sync_copy(lab_ref.at[pl.ds(0, _NKM * _N)], lab_v)

    def _zfill(q, c):
        cstg_v[pl.ds(_ZOFF + q * _KP, _KP)] = jnp.zeros((_KP,), jnp.int32)
        rstg_v[pl.ds(_ZOFF + q * _KP, _KP)] = jnp.zeros((_KP,), jnp.int32)
        return c
    lax.fori_loop(0, (_RSTR - _ZOFF) // _KP, _zfill, jnp.int32(0))
    for dsc in pend:
        dsc.wait()

    def _rbody(r, carry):
        rgv, totv = carry
        cols = knn_v[r]
        valid = cols < _N
        av = adjv_v[r // 8, pl.ds((r % 8) * _KP, _KP)]
        m = av != 0.0
        safe = jnp.where(valid, cols, 0)
        for s in range(_NKM):
            own = plsc.load_gather(lab_v, [rgv + (s * _N)])
            gl = plsc.load_gather(lab_v, [safe + (s * _N)])
            m = m | (gl == own)
        keep = valid & m
        key = jnp.where(keep, cols, jnp.int32(_SENT))
        skey, _ = plsc.sort_key_val(key, cols)
        cntv = plsc.all_reduce_population_count(keep)
        off = totv[0]
        cstg_v[pl.ds(off, _KP)] = skey
        rstg_v[pl.ds(off, _KP)] = rgv
        return (rgv + 1, totv + cntv)
    _, totv = lax.fori_loop(
        0, _RPW, _rbody, (base_rowv, jnp.zeros((_KP,), jnp.int32)))

    tot_v[...] = totv
    pltpu.sync_copy(cstg_v, interc_out.at[pl.ds(wid * _RSTR, _RSTR)])
    pltpu.sync_copy(rstg_v, interr_out.at[pl.ds(wid * _RSTR, _RSTR)])
    pltpu.sync_copy(tot_v, tot_out.at[wid])


def _srcidx_body(t_ref, o_ref):
    ti = t_ref[...]
    b = jnp.int32(0)
    bases = []
    for w in range(_NW):
        bases.append(b)
        b = b + ti[w, 0]
    m = b
    rows = lax.broadcasted_iota(jnp.int32, (_NW * _KP, 128), 0)
    cols = lax.broadcasted_iota(jnp.int32, (_NW * _KP, 128), 1)
    w16 = rows // _KP
    p = w16 * _OPW + (rows % _KP) * 128 + cols
    acc = p
    for v in range(_NW - 1):
        acc = acc + (p >= bases[v + 1]).astype(jnp.int32) * (_RSTR - ti[v, 0])
    zsrc = w16 * _RSTR + _ZOFF + (cols % 32)
    o_ref[...] = jnp.where(p >= m, zsrc, acc)


def _s2_body(interc_ref, interr_ref, src_ref, rows_out, cols_out,
             gidx_v, cval_v, rval_v, sem_s):
    wid = lax.axis_index("s") * 2 + lax.axis_index("c")
    pltpu.sync_copy(src_ref.at[pl.ds(wid * _KP, _KP)], gidx_v)

    pend = []
    for j in range(_OPW // 128):
        pend.append(pltpu.async_copy(
            interc_ref.at[gidx_v.at[j]], cval_v.at[j], sem_s))
        pend.append(pltpu.async_copy(
            interr_ref.at[gidx_v.at[j]], rval_v.at[j], sem_s))
        if len(pend) == 8:
            for dsc in pend:
                dsc.wait()
            pend = []
    for dsc in pend:
        dsc.wait()

    pltpu.sync_copy(cval_v, cols_out.at[wid])
    pltpu.sync_copy(rval_v, rows_out.at[wid])


def _run_simtopk(student, teacher):
    return pl.pallas_call(
        _simtopk_body,
        grid=(_N // _RB,),
        in_specs=[pl.BlockSpec((_RB, _D), lambda i: (i, 0)),
                  pl.BlockSpec((_N, _D), lambda i: (0, 0))],
        out_specs=pl.BlockSpec((_RB, _KP), lambda i: (i, 0)),
        out_shape=jax.ShapeDtypeStruct((_N, _KP), jnp.int32),
    )(student, teacher)


def _run_kmeans(teacher, teacher_t, cent0t):
    return pl.pallas_call(
        _kmeans_body,
        out_shape=jax.ShapeDtypeStruct((8, _N), jnp.int32),
    )(teacher, teacher_t, cent0t)


_SC_MESH = dict(core_axis_name="c", subcore_axis_name="s")


def _run_s1(adj_flat, knn, labs_flat, winfo):
    fn = pl.kernel(
        _s1_body,
        out_type=[jax.ShapeDtypeStruct((_NW * _RSTR,), jnp.int32),
                  jax.ShapeDtypeStruct((_NW * _RSTR,), jnp.int32),
                  jax.ShapeDtypeStruct((_NW, _KP), jnp.int32)],
        mesh=plsc.VectorSubcoreMesh(**_SC_MESH),
        compiler_params=pltpu.CompilerParams(needs_layout_passes=False),
        scratch_types=[
            pltpu.VMEM((_RPW, _KP), jnp.int32),
            pltpu.VMEM((_NKM * _N,), jnp.int32),
            pltpu.VMEM((_GCH, 128), jnp.int32),
            pltpu.VMEM((_GCH, 128), jnp.float32),
            pltpu.VMEM((_RSTR,), jnp.int32),
            pltpu.VMEM((_RSTR,), jnp.int32),
            pltpu.VMEM((_KP,), jnp.int32),
            pltpu.VMEM((_KP,), jnp.int32),
            pltpu.SemaphoreType.DMA,
        ],
    )
    return fn(adj_flat, knn, labs_flat, winfo)


def _run_srcidx(tots):
    return pl.pallas_call(
        _srcidx_body,
        out_shape=jax.ShapeDtypeStruct((_NW * _KP, 128), jnp.int32),
    )(tots)


def _run_s2(interc, interr, srcidx):
    fn = pl.kernel(
        _s2_body,
        out_type=[jax.ShapeDtypeStruct((_NW, _OPW // 128, 128), jnp.int32),
                  jax.ShapeDtypeStruct((_NW, _OPW // 128, 128), jnp.int32)],
        mesh=plsc.VectorSubcoreMesh(**_SC_MESH),
        compiler_params=pltpu.CompilerParams(needs_layout_passes=False),
        scratch_types=[
            pltpu.VMEM((_KP, 128), jnp.int32),
            pltpu.VMEM((_OPW // 128, 128), jnp.int32),
            pltpu.VMEM((_OPW // 128, 128), jnp.int32),
            pltpu.SemaphoreType.DMA,
        ],
    )
    return fn(interc, interr, srcidx)


def kernel(adj, student, teacher, top_k):
    tsg = lax.stop_gradient(teacher)
    knn = _run_simtopk(student, tsg)

    cents = []
    for s in range(_NKM):
        kk = jax.random.key(1234 + s)
        ii = jax.random.choice(kk, _N, shape=(_NC,), replace=False)
        cents.append(tsg[ii])
    cent0 = jnp.concatenate(cents, axis=0)
    labs = _run_kmeans(tsg, tsg.T, cent0.T)

    winfo = jnp.broadcast_to(
        jnp.arange(_NW, dtype=jnp.int32)[:, None], (_NW, _KP))
    interc, interr, tots = _run_s1(adj.reshape(-1), knn, labs.reshape(-1),
                                   winfo)
    srcidx = _run_srcidx(tots)
    rows_o, cols_o = _run_s2(interc, interr, srcidx)
    indices = jnp.stack([rows_o.reshape(-1), cols_o.reshape(-1)], axis=0)
    return (indices, top_k)


# speedup vs baseline: 1.2718x; 1.2481x over previous
import jax
import jax.numpy as jnp
from jax import lax
from jax.experimental import pallas as pl
from jax.experimental.pallas import tpu as pltpu
from jax.experimental.pallas import tpu_sc as plsc

_N = 4096
_D = 512
_K = 10
_KP = 16
_NC = 64
_NKM = 4
_NITER = 20
_RB = 256
_SENT = 2 ** 30

_NW = 32
_RPW = _N // _NW
_GCH = _RPW * _KP // 128
_ZPW = _N * _K // _NW
_ZCH = _ZPW // 128
_RSTR = 1344
_ZOFF = 1296
_OPW = _N * _K // _NW


def _simtopk_body(s_ref, t_ref, o_ref):
    i = pl.program_id(0)
    sim = lax.dot_general(s_ref[...], t_ref[...], (((1,), (1,)), ((), ())),
                          preferred_element_type=jnp.float32)
    row_g = i * _RB + lax.broadcasted_iota(jnp.int32, (_RB, _N), 0)
    col = lax.broadcasted_iota(jnp.int32, (_RB, _N), 1)
    sim = sim + jnp.where(col == row_g, jnp.float32(10.0), jnp.float32(0.0))
    idxs = []
    for _ in range(_K):
        idx = jnp.argmax(sim, axis=1).astype(jnp.int32)
        idxs.append(idx)
        sim = jnp.where(col == idx[:, None], -jnp.inf, sim)
    pad = jnp.full((_RB, _KP - _K), _SENT, jnp.int32)
    o_ref[...] = jnp.concatenate([jnp.stack(idxs, axis=1), pad], axis=1)


def _kmeans_body(x_ref, xt_ref, c_ref, o_ref):
    x = x_ref[...]
    xt = xt_ref[...]
    xx = jnp.sum(x * x, axis=1, keepdims=True)

    def _d2(centt):
        ip = lax.dot_general(x, centt, (((1,), (0,)), ((), ())),
                             preferred_element_type=jnp.float32)
        cc = jnp.sum(centt * centt, axis=0)
        return (xx - 2.0 * ip) + cc[None, :]

    def _step(_, centt):
        d2 = _d2(centt)
        h = jnp.concatenate(
            [(d2[:, s * _NC:(s + 1) * _NC]
              == jnp.min(d2[:, s * _NC:(s + 1) * _NC], axis=1, keepdims=True)
              ).astype(jnp.float32) for s in range(_NKM)], axis=1)
        sums_t = lax.dot_general(xt, h, (((1,), (0,)), ((), ())),
                                 preferred_element_type=jnp.float32)
        cnt = jnp.sum(h, axis=0)
        return sums_t / jnp.clip(cnt, 1.0)[None, :]

    centt = lax.fori_loop(0, _NITER, _step, c_ref[...])
    d2 = _d2(centt)
    for s in range(_NKM):
        o_ref[s, :] = jnp.argmin(
            d2[:, s * _NC:(s + 1) * _NC], axis=1).astype(jnp.int32)


def _sgather_body(adj_ref, knn_ref, winfo_ref, adjv_out,
                  knn_v, gidx_v, adjv_v, widb_v, sem_g):
    wid = lax.axis_index("s") * 2 + lax.axis_index("c")
    base_row = wid * _RPW
    pltpu.sync_copy(knn_ref.at[pl.ds(base_row, _RPW)], knn_v)
    pltpu.sync_copy(winfo_ref.at[wid], widb_v)
    base_rowv = widb_v[...] * _RPW

    def _gbody(r, gb):
        cols = knn_v[r]
        valid = cols < _N
        g = jnp.where(valid, gb + cols, 0)
        gidx_v[r // 8, pl.ds((r % 8) * _KP, _KP)] = g
        return gb + _N
    lax.fori_loop(0, _RPW, _gbody, base_rowv * _N)

    pend = [pltpu.async_copy(adj_ref.at[gidx_v.at[j]], adjv_v.at[j], sem_g)
            for j in range(_GCH)]
    for dsc in pend:
        dsc.wait()
    pltpu.sync_copy(adjv_v, adjv_out.at[wid])


def _s1_body(adjv_ref, knn_ref, lab_ref, winfo_ref, interc_out, interr_out,
             tot_out, knn_v, lab_v, adjv_v, cstg_v, rstg_v, widb_v,
             tot_v):
    wid = lax.axis_index("s") * 2 + lax.axis_index("c")
    base_row = wid * _RPW
    pltpu.sync_copy(knn_ref.at[pl.ds(base_row, _RPW)], knn_v)
    pltpu.sync_copy(winfo_ref.at[wid], widb_v)
    base_rowv = widb_v[...] * _RPW
    pltpu.sync_copy(adjv_ref.at[wid], adjv_v)
    pltpu.sync_copy(lab_ref.at[pl.ds(0, _NKM * _N)], lab_v)

    def _zfill(q, c):
        cstg_v[pl.ds(_ZOFF + q * _KP, _KP)] = jnp.zeros((_KP,), jnp.int32)
        rstg_v[pl.ds(_ZOFF + q * _KP, _KP)] = jnp.zeros((_KP,), jnp.int32)
        return c
    lax.fori_loop(0, (_RSTR - _ZOFF) // _KP, _zfill, jnp.int32(0))

    def _rbody(r, carry):
        rgv, totv = carry
        cols = knn_v[r]
        valid = cols < _N
        av = adjv_v[r // 8, pl.ds((r % 8) * _KP, _KP)]
        m = av != 0.0
        safe = jnp.where(valid, cols, 0)
        for s in range(_NKM):
            own = plsc.load_gather(lab_v, [rgv + (s * _N)])
            gl = plsc.load_gather(lab_v, [safe + (s * _N)])
            m = m | (gl == own)
        keep = valid & m
        key = jnp.where(keep, cols, jnp.int32(_SENT))
        skey, _ = plsc.sort_key_val(key, cols)
        cntv = plsc.all_reduce_population_count(keep)
        off = totv[0]
        cstg_v[pl.ds(off, _KP)] = skey
        rstg_v[pl.ds(off, _KP)] = rgv
        return (rgv + 1, totv + cntv)
    _, totv = lax.fori_loop(
        0, _RPW, _rbody, (base_rowv, jnp.zeros((_KP,), jnp.int32)))

    tot_v[...] = totv
    pltpu.sync_copy(cstg_v, interc_out.at[pl.ds(wid * _RSTR, _RSTR)])
    pltpu.sync_copy(rstg_v, interr_out.at[pl.ds(wid * _RSTR, _RSTR)])
    pltpu.sync_copy(tot_v, tot_out.at[wid])


def _srcidx_body(t_ref, o_ref):
    ti = t_ref[...]
    b = jnp.int32(0)
    bases = []
    for w in range(_NW):
        bases.append(b)
        b = b + ti[w, 0]
    m = b
    rows = lax.broadcasted_iota(jnp.int32, (_NW * _KP, 128), 0)
    cols = lax.broadcasted_iota(jnp.int32, (_NW * _KP, 128), 1)
    w16 = rows // _KP
    p = w16 * _OPW + (rows % _KP) * 128 + cols
    acc = p
    for v in range(_NW - 1):
        acc = acc + (p >= bases[v + 1]).astype(jnp.int32) * (_RSTR - ti[v, 0])
    zsrc = w16 * _RSTR + _ZOFF + (cols % 32)
    o_ref[...] = jnp.where(p >= m, zsrc, acc)


def _s2_body(interc_ref, interr_ref, src_ref, rows_out, cols_out,
             gidx_v, cval_v, rval_v, sem_s):
    wid = lax.axis_index("s") * 2 + lax.axis_index("c")
    pltpu.sync_copy(src_ref.at[pl.ds(wid * _KP, _KP)], gidx_v)

    pend = []
    for j in range(_OPW // 128):
        pend.append(pltpu.async_copy(
            interc_ref.at[gidx_v.at[j]], cval_v.at[j], sem_s))
        pend.append(pltpu.async_copy(
            interr_ref.at[gidx_v.at[j]], rval_v.at[j], sem_s))
        if len(pend) == 8:
            for dsc in pend:
                dsc.wait()
            pend = []
    for dsc in pend:
        dsc.wait()

    pltpu.sync_copy(cval_v, cols_out.at[wid])
    pltpu.sync_copy(rval_v, rows_out.at[wid])


def _run_simtopk(student, teacher):
    return pl.pallas_call(
        _simtopk_body,
        grid=(_N // _RB,),
        in_specs=[pl.BlockSpec((_RB, _D), lambda i: (i, 0)),
                  pl.BlockSpec((_N, _D), lambda i: (0, 0))],
        out_specs=pl.BlockSpec((_RB, _KP), lambda i: (i, 0)),
        out_shape=jax.ShapeDtypeStruct((_N, _KP), jnp.int32),
    )(student, teacher)


def _run_kmeans(teacher, teacher_t, cent0t):
    return pl.pallas_call(
        _kmeans_body,
        out_shape=jax.ShapeDtypeStruct((8, _N), jnp.int32),
    )(teacher, teacher_t, cent0t)


_SC_MESH = dict(core_axis_name="c", subcore_axis_name="s")


def _run_sgather(adj_flat, knn, winfo):
    fn = pl.kernel(
        _sgather_body,
        out_type=[jax.ShapeDtypeStruct((_NW, _GCH, 128), jnp.float32)],
        mesh=plsc.VectorSubcoreMesh(**_SC_MESH),
        compiler_params=pltpu.CompilerParams(needs_layout_passes=False),
        scratch_types=[
            pltpu.VMEM((_RPW, _KP), jnp.int32),
            pltpu.VMEM((_GCH, 128), jnp.int32),
            pltpu.VMEM((_GCH, 128), jnp.float32),
            pltpu.VMEM((_KP,), jnp.int32),
            pltpu.SemaphoreType.DMA,
        ],
    )
    return fn(adj_flat, knn, winfo)[0]


def _run_s1(adjv, knn, labs_flat, winfo):
    fn = pl.kernel(
        _s1_body,
        out_type=[jax.ShapeDtypeStruct((_NW * _RSTR,), jnp.int32),
                  jax.ShapeDtypeStruct((_NW * _RSTR,), jnp.int32),
                  jax.ShapeDtypeStruct((_NW, _KP), jnp.int32)],
        mesh=plsc.VectorSubcoreMesh(**_SC_MESH),
        compiler_params=pltpu.CompilerParams(needs_layout_passes=False),
        scratch_types=[
            pltpu.VMEM((_RPW, _KP), jnp.int32),
            pltpu.VMEM((_NKM * _N,), jnp.int32),
            pltpu.VMEM((_GCH, 128), jnp.float32),
            pltpu.VMEM((_RSTR,), jnp.int32),
            pltpu.VMEM((_RSTR,), jnp.int32),
            pltpu.VMEM((_KP,), jnp.int32),
            pltpu.VMEM((_KP,), jnp.int32),
        ],
    )
    return fn(adjv, knn, labs_flat, winfo)


def _run_srcidx(tots):
    return pl.pallas_call(
        _srcidx_body,
        out_shape=jax.ShapeDtypeStruct((_NW * _KP, 128), jnp.int32),
    )(tots)


def _run_s2(interc, interr, srcidx):
    fn = pl.kernel(
        _s2_body,
        out_type=[jax.ShapeDtypeStruct((_NW, _OPW // 128, 128), jnp.int32),
                  jax.ShapeDtypeStruct((_NW, _OPW // 128, 128), jnp.int32)],
        mesh=plsc.VectorSubcoreMesh(**_SC_MESH),
        compiler_params=pltpu.CompilerParams(needs_layout_passes=False),
        scratch_types=[
            pltpu.VMEM((_KP, 128), jnp.int32),
            pltpu.VMEM((_OPW // 128, 128), jnp.int32),
            pltpu.VMEM((_OPW // 128, 128), jnp.int32),
            pltpu.SemaphoreType.DMA,
        ],
    )
    return fn(interc, interr, srcidx)


def kernel(adj, student, teacher, top_k):
    tsg = lax.stop_gradient(teacher)
    knn = _run_simtopk(student, tsg)

    winfo = jnp.broadcast_to(
        jnp.arange(_NW, dtype=jnp.int32)[:, None], (_NW, _KP))
    adjv = _run_sgather(adj.reshape(-1), knn, winfo)

    cents = []
    for s in range(_NKM):
        kk = jax.random.key(1234 + s)
        ii = jax.random.choice(kk, _N, shape=(_NC,), replace=False)
        cents.append(tsg[ii])
    cent0 = jnp.concatenate(cents, axis=0)
    labs = _run_kmeans(tsg, tsg.T, cent0.T)

    interc, interr, tots = _run_s1(adjv, knn, labs.reshape(-1), winfo)
    srcidx = _run_srcidx(tots)
    rows_o, cols_o = _run_s2(interc, interr, srcidx)
    indices = jnp.stack([rows_o.reshape(-1), cols_o.reshape(-1)], axis=0)
    return (indices, top_k)
